# baseline (speedup 1.0000x reference)
import functools

import jax
import jax.numpy as jnp
from jax import lax
from jax.experimental import pallas as pl
from jax.experimental.pallas import tpu as pltpu
from jax.experimental.pallas import tpu_sc as plsc

N = 10000
D = 128
NC = 2
NS = 16
NW = NC * NS
NP = 10240
NPAD_ROWS = NP - N
RPS = NP // NS

E = 320000
KE = 80
E2 = 160
CW = 64
EP = NW * KE * 128

Q = 100000
KQ = 52
QC = 64
QP = NW * KQ * QC


_GATHER_DNUMS = lax.GatherDimensionNumbers(
    offset_dims=(), collapsed_slice_dims=(0,), start_index_map=(0,))


def _lane_shuffle(v, idx):
    return lax.gather(v, idx[:, None], _GATHER_DNUMS, slice_sizes=(1,),
                      mode=lax.GatherScatterMode.PROMISE_IN_BOUNDS)


def _mesh():
    return plsc.VectorSubcoreMesh(
        core_axis_name="c", subcore_axis_name="s",
        num_cores=NC, num_subcores=NS)


def _sc_degree(dst3, z16):
    @functools.partial(
        pl.kernel,
        out_type=jax.ShapeDtypeStruct((NC, NP, 16), jnp.float32),
        mesh=_mesh(),
        scratch_types=[
            pltpu.VMEM((KE, 128), jnp.int32),
            pltpu.VMEM((128, 16), jnp.float32),
            pltpu.VMEM_SHARED((NP, 16), jnp.float32),
        ],
    )
    def k(dst_hbm, z_hbm, out_hbm, didx, ones_v, acc):
        c = lax.axis_index("c")
        s = lax.axis_index("s")
        w = s * NC + c

        def fill(i, carry):
            ones_v[i] = jnp.ones((16,), jnp.float32)
            return carry
        lax.fori_loop(0, 128, fill, 0)

        pltpu.sync_copy(z_hbm.at[pl.ds(s * RPS, RPS)],
                        acc.at[pl.ds(s * RPS, RPS)])
        pltpu.sync_copy(dst_hbm.at[w], didx)
        plsc.subcore_barrier()

        def body(j, carry):
            pltpu.sync_copy(ones_v, acc.at[didx.at[j]], add=True)
            return carry
        lax.fori_loop(0, KE, body, 0)

        plsc.subcore_barrier()
        pltpu.sync_copy(acc.at[pl.ds(s * RPS, RPS)],
                        out_hbm.at[c, pl.ds(s * RPS, RPS)])

    return k(dst3, z16)


def _sc_aggregate(hs, sd3, zrows):
    @functools.partial(
        pl.kernel,
        out_type=jax.ShapeDtypeStruct((NC, NP, D), jnp.float32),
        mesh=_mesh(),
        scratch_types=[
            pltpu.VMEM((E2 // 2, 128), jnp.int32),
            pltpu.VMEM((CW, D), jnp.float32),
            pltpu.VMEM((CW, D), jnp.float32),
            pltpu.VMEM((CW, D), jnp.float32),
            pltpu.VMEM((CW, D), jnp.float32),
            pltpu.VMEM_SHARED((NP, D), jnp.float32),
            pltpu.SemaphoreType.DMA,
            pltpu.SemaphoreType.DMA,
            pltpu.SemaphoreType.DMA,
            pltpu.SemaphoreType.DMA,
            pltpu.SemaphoreType.DMA,
            pltpu.SemaphoreType.DMA,
            pltpu.SemaphoreType.DMA,
            pltpu.SemaphoreType.DMA,
        ],
    )
    def k(hs_hbm, sd_hbm, z_hbm, out_hbm,
          sd, r0, r1, r2, r3, acc,
          g0, g1, g2, g3, s0, s1, s2, s3):
        rows = (r0, r1, r2, r3)
        gsem = (g0, g1, g2, g3)
        ssem = (s0, s1, s2, s3)
        c = lax.axis_index("c")
        s = lax.axis_index("s")
        w = s * NC + c

        pltpu.sync_copy(z_hbm.at[pl.ds(s * RPS, RPS)],
                        acc.at[pl.ds(s * RPS, RPS)])
        plsc.subcore_barrier()

        PH = E2 // 2

        def gather(j, b):
            pltpu.async_copy(hs_hbm.at[sd.at[j, pl.ds(0, CW)]], rows[b], gsem[b])

        def gwait(j, b):
            pltpu.make_async_copy(
                hs_hbm.at[sd.at[j, pl.ds(0, CW)]], rows[b], gsem[b]).wait()

        def scat(j, b):
            pltpu.async_copy(rows[b], acc.at[sd.at[j, pl.ds(CW, CW)]],
                             ssem[b], add=True)

        def swait(j, b):
            pltpu.make_async_copy(
                rows[b], acc.at[sd.at[j, pl.ds(CW, CW)]], ssem[b]).wait()

        def phase(p, carry):
            pltpu.sync_copy(sd_hbm.at[w, pl.ds(p * PH, PH)], sd)
            gather(0, 0)
            gather(1, 1)

            def quad(i, c2):
                j0 = i * 4
                for kk in range(4):
                    j = j0 + kk
                    gwait(j, kk)
                    scat(j, kk)
                    bn = (kk + 2) % 4

                    @pl.when(j + 2 < PH)
                    def _prefetch():
                        @pl.when(j >= 2)
                        def _free():
                            swait(j - 2, bn)
                        gather(j + 2, bn)
                return c2
            lax.fori_loop(0, PH // 4, quad, 0)
            for kk in range(4):
                swait(PH - 4 + kk, kk)
            return carry
        lax.fori_loop(0, 2, phase, 0)

        plsc.subcore_barrier()
        pltpu.sync_copy(acc.at[pl.ds(s * RPS, RPS)],
                        out_hbm.at[c, pl.ds(s * RPS, RPS)])

    return k(hs, sd3, zrows)


def _sc_score(h3, qs3, qt3):
    @functools.partial(
        pl.kernel,
        out_type=jax.ShapeDtypeStruct((QP,), jnp.float32),
        mesh=_mesh(),
        scratch_types=[
            pltpu.VMEM((KQ, QC), jnp.int32),
            pltpu.VMEM((KQ, QC), jnp.int32),
            pltpu.VMEM((QC, D), jnp.float32),
            pltpu.VMEM((QC, D), jnp.float32),
            pltpu.VMEM((QC, D), jnp.float32),
            pltpu.VMEM((QC, D), jnp.float32),
            pltpu.VMEM((QC,), jnp.float32),
            pltpu.SemaphoreType.DMA,
            pltpu.SemaphoreType.DMA,
        ],
    )
    def k(h3_hbm, qs_hbm, qt_hbm, out_hbm,
          qsi, qti, xs0, xt0, xs1, xt1, sbuf, sem0, sem1):
        c = lax.axis_index("c")
        s = lax.axis_index("s")
        w = s * NC + c
        base = w * (KQ * QC)
        lanes = lax.broadcasted_iota(jnp.int32, (16,), 0)

        pltpu.sync_copy(qs_hbm.at[w], qsi)
        pltpu.sync_copy(qt_hbm.at[w], qti)

        def start(j, xs, xt, sem):
            pltpu.async_copy(h3_hbm.at[qsi.at[j]], xs, sem)
            pltpu.async_copy(h3_hbm.at[qti.at[j]], xt, sem)

        def drain(j, xs, xt, sem):
            pltpu.make_async_copy(h3_hbm.at[qsi.at[j]], xs, sem).wait()
            pltpu.make_async_copy(h3_hbm.at[qti.at[j]], xt, sem).wait()

        def compute(j, xs, xt):
            def group(g, carry2):
                svec = jnp.zeros((16,), jnp.float32)
                for l in range(16):
                    i = g * 16 + l
                    acc = xs[i, pl.ds(0, 16)] * xt[i, pl.ds(0, 16)]
                    for kk in range(1, 8):
                        acc = acc + (xs[i, pl.ds(kk * 16, 16)]
                                     * xt[i, pl.ds(kk * 16, 16)])
                    for m in (1, 2, 4, 8):
                        acc = acc + _lane_shuffle(acc, lanes ^ m)
                    svec = jnp.where(lanes == l, acc, svec)
                sbuf[pl.ds(g * 16, 16)] = 1.0 / (1.0 + jnp.exp(-svec))
                return carry2
            lax.fori_loop(0, QC // 16, group, 0)
            pltpu.sync_copy(sbuf, out_hbm.at[pl.ds(base + j * QC, QC)])

        H = KQ // 2
        start(0, xs0, xt0, sem0)

        def chunk(jj, carry):
            j0 = jj * 2
            start(j0 + 1, xs1, xt1, sem1)
            drain(j0, xs0, xt0, sem0)
            compute(j0, xs0, xt0)

            @pl.when(jj < H - 1)
            def _prefetch():
                start(j0 + 2, xs0, xt0, sem0)

            drain(j0 + 1, xs1, xt1, sem1)
            compute(j0 + 1, xs1, xt1)
            return carry
        lax.fori_loop(0, H, chunk, 0)

    return k(h3, qs3, qt3)


def _dinv_from(degp0, degp1):
    return lax.rsqrt(degp0[:, 0:1] + degp1[:, 0:1] + 1.0)


def _tc_first(x_p, W1, degp):
    def body(x_ref, w_ref, degp_ref, out_ref):
        dinv = _dinv_from(degp_ref[0], degp_ref[1])
        h = jnp.dot(x_ref[...], w_ref[...], preferred_element_type=jnp.float32)
        out_ref[...] = h * dinv

    return pl.pallas_call(
        body,
        out_shape=jax.ShapeDtypeStruct((NP, D), jnp.float32),
    )(x_p, W1, degp)


def _tc_mid(aggp, hs, degp, b, g, be, Wn):
    def body(aggp_ref, hs_ref, degp_ref, b_ref, g_ref, be_ref, w_ref, out_ref):
        dinv = _dinv_from(degp_ref[0], degp_ref[1])
        pre = (aggp_ref[0] + aggp_ref[1] + hs_ref[...]) * dinv + b_ref[...]
        valid = pre[0:N, :]
        m = jnp.mean(valid, axis=0, keepdims=True)
        v = jnp.mean((valid - m) ** 2, axis=0, keepdims=True)
        hbn = (pre - m) * lax.rsqrt(v + 1e-5) * g_ref[...] + be_ref[...]
        hr = jnp.maximum(hbn, 0.0)
        hn = jnp.dot(hr, w_ref[...], preferred_element_type=jnp.float32) * dinv
        rows = lax.broadcasted_iota(jnp.int32, (NP, 1), 0)
        out_ref[...] = jnp.where(rows < N, hn, 0.0)

    return pl.pallas_call(
        body,
        out_shape=jax.ShapeDtypeStruct((NP, D), jnp.float32),
    )(aggp, hs, degp, b, g, be, Wn)


def _tc_final(aggp, hs, degp, b):
    def body(aggp_ref, hs_ref, degp_ref, b_ref, out_ref):
        dinv = _dinv_from(degp_ref[0], degp_ref[1])
        pre = (aggp_ref[0] + aggp_ref[1] + hs_ref[...]) * dinv + b_ref[...]
        rows = lax.broadcasted_iota(jnp.int32, (NP, 1), 0)
        out_ref[...] = jnp.where(rows < N, pre, 0.0)

    return pl.pallas_call(
        body,
        out_shape=jax.ShapeDtypeStruct((NP, D), jnp.float32),
    )(aggp, hs, degp, b)


def _pad_idx(idx, total):
    pad = total - idx.shape[0]
    fill = N + (jnp.arange(pad, dtype=jnp.int32) % NPAD_ROWS)
    return jnp.concatenate([idx.astype(jnp.int32), fill])


def kernel(x, edge_index, edges, W1, b1, W2, b2, W3, b3, g1, be1, g2, be2):
    src3 = _pad_idx(edge_index[0], EP).reshape(NW, E2, 1, CW)
    dst3 = _pad_idx(edge_index[1], EP).reshape(NW, E2, 1, CW)
    sd3 = jnp.concatenate([src3, dst3], axis=2).reshape(NW, E2, 128)
    dst3_deg = dst3.reshape(NW, KE, 128)
    qs3 = _pad_idx(edges[0], QP).reshape(NW, KQ, QC)
    qt3 = _pad_idx(edges[1], QP).reshape(NW, KQ, QC)

    x_p = jnp.zeros((NP, D), jnp.float32).at[:N].set(x)
    zrows = jnp.zeros((NP, D), jnp.float32)
    z16 = jnp.zeros((NP, 16), jnp.float32)
    b1r, b2r, b3r = (v.reshape(1, D) for v in (b1, b2, b3))
    g1r, g2r = g1.reshape(1, D), g2.reshape(1, D)
    be1r, be2r = be1.reshape(1, D), be2.reshape(1, D)

    degp = _sc_degree(dst3_deg, z16)

    hs1 = _tc_first(x_p, W1, degp)
    agg1 = _sc_aggregate(hs1, sd3, zrows)
    hs2 = _tc_mid(agg1, hs1, degp, b1r, g1r, be1r, W2)
    agg2 = _sc_aggregate(hs2, sd3, zrows)
    hs3 = _tc_mid(agg2, hs2, degp, b2r, g2r, be2r, W3)
    agg3 = _sc_aggregate(hs3, sd3, zrows)
    h3 = _tc_final(agg3, hs3, degp, b3r)

    scores = _sc_score(h3, qs3, qt3)
    return scores[:Q]


# speedup vs baseline: 1.0708x; 1.0708x over previous
import functools

import jax
import jax.numpy as jnp
from jax import lax
from jax.experimental import pallas as pl
from jax.experimental.pallas import tpu as pltpu
from jax.experimental.pallas import tpu_sc as plsc

N = 10000
D = 128
NC = 2
NS = 16
NW = NC * NS
NP = 10240
NPAD_ROWS = NP - N
RPS = NP // NS

E = 320000
KE = 80
E2 = 80
CW = 128
EP = NW * KE * 128

Q = 100000
KQ = 52
QC = 64
QP = NW * KQ * QC


_GATHER_DNUMS = lax.GatherDimensionNumbers(
    offset_dims=(), collapsed_slice_dims=(0,), start_index_map=(0,))


def _lane_shuffle(v, idx):
    return lax.gather(v, idx[:, None], _GATHER_DNUMS, slice_sizes=(1,),
                      mode=lax.GatherScatterMode.PROMISE_IN_BOUNDS)


def _mesh():
    return plsc.VectorSubcoreMesh(
        core_axis_name="c", subcore_axis_name="s",
        num_cores=NC, num_subcores=NS)


def _sc_degree(dst3, z16):
    @functools.partial(
        pl.kernel,
        out_type=jax.ShapeDtypeStruct((NC, NP, 16), jnp.float32),
        mesh=_mesh(),
        scratch_types=[
            pltpu.VMEM((KE, 128), jnp.int32),
            pltpu.VMEM((128, 16), jnp.float32),
            pltpu.VMEM_SHARED((NP, 16), jnp.float32),
        ],
    )
    def k(dst_hbm, z_hbm, out_hbm, didx, ones_v, acc):
        c = lax.axis_index("c")
        s = lax.axis_index("s")
        w = s * NC + c

        def fill(i, carry):
            ones_v[i] = jnp.ones((16,), jnp.float32)
            return carry
        lax.fori_loop(0, 128, fill, 0)

        pltpu.sync_copy(z_hbm.at[pl.ds(s * RPS, RPS)],
                        acc.at[pl.ds(s * RPS, RPS)])
        pltpu.sync_copy(dst_hbm.at[w], didx)
        plsc.subcore_barrier()

        def body(j, carry):
            pltpu.sync_copy(ones_v, acc.at[didx.at[j]], add=True)
            return carry
        lax.fori_loop(0, KE, body, 0)

        plsc.subcore_barrier()
        pltpu.sync_copy(acc.at[pl.ds(s * RPS, RPS)],
                        out_hbm.at[c, pl.ds(s * RPS, RPS)])

    return k(dst3, z16)


def _sc_aggregate(hs, sd3, zrows):
    @functools.partial(
        pl.kernel,
        out_type=jax.ShapeDtypeStruct((NC, NP, D), jnp.float32),
        mesh=_mesh(),
        scratch_types=[
            pltpu.VMEM((E2 // 2, 256), jnp.int32),
            pltpu.VMEM((CW, D), jnp.float32),
            pltpu.VMEM((CW, D), jnp.float32),
            pltpu.VMEM_SHARED((NP, D), jnp.float32),
            pltpu.SemaphoreType.DMA,
            pltpu.SemaphoreType.DMA,
            pltpu.SemaphoreType.DMA,
            pltpu.SemaphoreType.DMA,
        ],
    )
    def k(hs_hbm, sd_hbm, z_hbm, out_hbm,
          sd, r0, r1, acc, g0, g1, s0, s1):
        rows = (r0, r1)
        gsem = (g0, g1)
        ssem = (s0, s1)
        c = lax.axis_index("c")
        s = lax.axis_index("s")
        w = s * NC + c

        pltpu.sync_copy(z_hbm.at[pl.ds(s * RPS, RPS)],
                        acc.at[pl.ds(s * RPS, RPS)])
        plsc.subcore_barrier()

        PH = E2 // 2

        def gather(j, b):
            pltpu.async_copy(hs_hbm.at[sd.at[j, pl.ds(0, CW)]], rows[b], gsem[b])

        def gwait(j, b):
            pltpu.make_async_copy(
                hs_hbm.at[sd.at[j, pl.ds(0, CW)]], rows[b], gsem[b]).wait()

        def scat(j, b):
            pltpu.async_copy(rows[b], acc.at[sd.at[j, pl.ds(CW, CW)]],
                             ssem[b], add=True)

        def swait(j, b):
            pltpu.make_async_copy(
                rows[b], acc.at[sd.at[j, pl.ds(CW, CW)]], ssem[b]).wait()

        def phase(p, carry):
            pltpu.sync_copy(sd_hbm.at[w, pl.ds(p * PH, PH)], sd)
            gather(0, 0)
            gather(1, 1)

            def duo(i, c2):
                j0 = i * 2
                for kk in range(2):
                    j = j0 + kk
                    gwait(j, kk)
                    scat(j, kk)

                    @pl.when(j + 2 < PH)
                    def _prefetch():
                        swait(j, kk)
                        gather(j + 2, kk)
                return c2
            lax.fori_loop(0, PH // 2, duo, 0)
            swait(PH - 2, 0)
            swait(PH - 1, 1)
            return carry
        lax.fori_loop(0, 2, phase, 0)

        plsc.subcore_barrier()
        pltpu.sync_copy(acc.at[pl.ds(s * RPS, RPS)],
                        out_hbm.at[c, pl.ds(s * RPS, RPS)])

    return k(hs, sd3, zrows)


def _sc_score(h3, qs3, qt3):
    @functools.partial(
        pl.kernel,
        out_type=jax.ShapeDtypeStruct((QP,), jnp.float32),
        mesh=_mesh(),
        scratch_types=[
            pltpu.VMEM((KQ, QC), jnp.int32),
            pltpu.VMEM((KQ, QC), jnp.int32),
            pltpu.VMEM((QC, D), jnp.float32),
            pltpu.VMEM((QC, D), jnp.float32),
            pltpu.VMEM((QC, D), jnp.float32),
            pltpu.VMEM((QC, D), jnp.float32),
            pltpu.VMEM((QC,), jnp.float32),
            pltpu.SemaphoreType.DMA,
            pltpu.SemaphoreType.DMA,
        ],
    )
    def k(h3_hbm, qs_hbm, qt_hbm, out_hbm,
          qsi, qti, xs0, xt0, xs1, xt1, sbuf, sem0, sem1):
        c = lax.axis_index("c")
        s = lax.axis_index("s")
        w = s * NC + c
        base = w * (KQ * QC)
        lanes = lax.broadcasted_iota(jnp.int32, (16,), 0)

        pltpu.sync_copy(qs_hbm.at[w], qsi)
        pltpu.sync_copy(qt_hbm.at[w], qti)

        def start(j, xs, xt, sem):
            pltpu.async_copy(h3_hbm.at[qsi.at[j]], xs, sem)
            pltpu.async_copy(h3_hbm.at[qti.at[j]], xt, sem)

        def drain(j, xs, xt, sem):
            pltpu.make_async_copy(h3_hbm.at[qsi.at[j]], xs, sem).wait()
            pltpu.make_async_copy(h3_hbm.at[qti.at[j]], xt, sem).wait()

        def compute(j, xs, xt):
            def group(g, carry2):
                svec = jnp.zeros((16,), jnp.float32)
                for l in range(16):
                    i = g * 16 + l
                    acc = xs[i, pl.ds(0, 16)] * xt[i, pl.ds(0, 16)]
                    for kk in range(1, 8):
                        acc = acc + (xs[i, pl.ds(kk * 16, 16)]
                                     * xt[i, pl.ds(kk * 16, 16)])
                    for m in (1, 2, 4, 8):
                        acc = acc + _lane_shuffle(acc, lanes ^ m)
                    svec = jnp.where(lanes == l, acc, svec)
                sbuf[pl.ds(g * 16, 16)] = 1.0 / (1.0 + jnp.exp(-svec))
                return carry2
            lax.fori_loop(0, QC // 16, group, 0)
            pltpu.sync_copy(sbuf, out_hbm.at[pl.ds(base + j * QC, QC)])

        H = KQ // 2
        start(0, xs0, xt0, sem0)

        def chunk(jj, carry):
            j0 = jj * 2
            start(j0 + 1, xs1, xt1, sem1)
            drain(j0, xs0, xt0, sem0)
            compute(j0, xs0, xt0)

            @pl.when(jj < H - 1)
            def _prefetch():
                start(j0 + 2, xs0, xt0, sem0)

            drain(j0 + 1, xs1, xt1, sem1)
            compute(j0 + 1, xs1, xt1)
            return carry
        lax.fori_loop(0, H, chunk, 0)

    return k(h3, qs3, qt3)


def _dinv_from(degp0, degp1):
    return lax.rsqrt(degp0[:, 0:1] + degp1[:, 0:1] + 1.0)


def _tc_first(x_p, W1, degp):
    def body(x_ref, w_ref, degp_ref, out_ref):
        dinv = _dinv_from(degp_ref[0], degp_ref[1])
        h = jnp.dot(x_ref[...], w_ref[...], preferred_element_type=jnp.float32)
        out_ref[...] = h * dinv

    return pl.pallas_call(
        body,
        out_shape=jax.ShapeDtypeStruct((NP, D), jnp.float32),
    )(x_p, W1, degp)


def _tc_mid(aggp, hs, degp, b, g, be, Wn):
    def body(aggp_ref, hs_ref, degp_ref, b_ref, g_ref, be_ref, w_ref, out_ref):
        dinv = _dinv_from(degp_ref[0], degp_ref[1])
        pre = (aggp_ref[0] + aggp_ref[1] + hs_ref[...]) * dinv + b_ref[...]
        valid = pre[0:N, :]
        m = jnp.mean(valid, axis=0, keepdims=True)
        v = jnp.mean((valid - m) ** 2, axis=0, keepdims=True)
        hbn = (pre - m) * lax.rsqrt(v + 1e-5) * g_ref[...] + be_ref[...]
        hr = jnp.maximum(hbn, 0.0)
        hn = jnp.dot(hr, w_ref[...], preferred_element_type=jnp.float32) * dinv
        rows = lax.broadcasted_iota(jnp.int32, (NP, 1), 0)
        out_ref[...] = jnp.where(rows < N, hn, 0.0)

    return pl.pallas_call(
        body,
        out_shape=jax.ShapeDtypeStruct((NP, D), jnp.float32),
    )(aggp, hs, degp, b, g, be, Wn)


def _tc_final(aggp, hs, degp, b):
    def body(aggp_ref, hs_ref, degp_ref, b_ref, out_ref):
        dinv = _dinv_from(degp_ref[0], degp_ref[1])
        pre = (aggp_ref[0] + aggp_ref[1] + hs_ref[...]) * dinv + b_ref[...]
        rows = lax.broadcasted_iota(jnp.int32, (NP, 1), 0)
        out_ref[...] = jnp.where(rows < N, pre, 0.0)

    return pl.pallas_call(
        body,
        out_shape=jax.ShapeDtypeStruct((NP, D), jnp.float32),
    )(aggp, hs, degp, b)


def _pad_idx(idx, total):
    pad = total - idx.shape[0]
    fill = N + (jnp.arange(pad, dtype=jnp.int32) % NPAD_ROWS)
    return jnp.concatenate([idx.astype(jnp.int32), fill])


def kernel(x, edge_index, edges, W1, b1, W2, b2, W3, b3, g1, be1, g2, be2):
    src3 = _pad_idx(edge_index[0], EP).reshape(NW, E2, 1, CW)
    dst3 = _pad_idx(edge_index[1], EP).reshape(NW, E2, 1, CW)
    sd3 = jnp.concatenate([src3, dst3], axis=2).reshape(NW, E2, 2 * CW)
    dst3_deg = dst3.reshape(NW, KE, 128)
    qs3 = _pad_idx(edges[0], QP).reshape(NW, KQ, QC)
    qt3 = _pad_idx(edges[1], QP).reshape(NW, KQ, QC)

    x_p = jnp.zeros((NP, D), jnp.float32).at[:N].set(x)
    zrows = jnp.zeros((NP, D), jnp.float32)
    z16 = jnp.zeros((NP, 16), jnp.float32)
    b1r, b2r, b3r = (v.reshape(1, D) for v in (b1, b2, b3))
    g1r, g2r = g1.reshape(1, D), g2.reshape(1, D)
    be1r, be2r = be1.reshape(1, D), be2.reshape(1, D)

    degp = _sc_degree(dst3_deg, z16)

    hs1 = _tc_first(x_p, W1, degp)
    agg1 = _sc_aggregate(hs1, sd3, zrows)
    hs2 = _tc_mid(agg1, hs1, degp, b1r, g1r, be1r, W2)
    agg2 = _sc_aggregate(hs2, sd3, zrows)
    hs3 = _tc_mid(agg2, hs2, degp, b2r, g2r, be2r, W3)
    agg3 = _sc_aggregate(hs3, sd3, zrows)
    h3 = _tc_final(agg3, hs3, degp, b3r)

    scores = _sc_score(h3, qs3, qt3)
    return scores[:Q]


# speedup vs baseline: 1.0724x; 1.0014x over previous
import functools

import jax
import jax.numpy as jnp
from jax import lax
from jax.experimental import pallas as pl
from jax.experimental.pallas import tpu as pltpu
from jax.experimental.pallas import tpu_sc as plsc

N = 10000
D = 128
NC = 2
NS = 16
NW = NC * NS
NP = 10240
NPAD_ROWS = NP - N
RPS = NP // NS

E = 320000
KE = 80
E2 = 80
CW = 128
EP = NW * KE * 128

Q = 100000
KQ = 52
QC = 64
QP = NW * KQ * QC


_GATHER_DNUMS = lax.GatherDimensionNumbers(
    offset_dims=(), collapsed_slice_dims=(0,), start_index_map=(0,))


def _lane_shuffle(v, idx):
    return lax.gather(v, idx[:, None], _GATHER_DNUMS, slice_sizes=(1,),
                      mode=lax.GatherScatterMode.PROMISE_IN_BOUNDS)


def _mesh():
    return plsc.VectorSubcoreMesh(
        core_axis_name="c", subcore_axis_name="s",
        num_cores=NC, num_subcores=NS)


def _sc_degree(dst3, z16):
    @functools.partial(
        pl.kernel,
        out_type=jax.ShapeDtypeStruct((NC, NP, 16), jnp.float32),
        mesh=_mesh(),
        scratch_types=[
            pltpu.VMEM((KE, 128), jnp.int32),
            pltpu.VMEM((128, 16), jnp.float32),
            pltpu.VMEM_SHARED((NP, 16), jnp.float32),
        ],
    )
    def k(dst_hbm, z_hbm, out_hbm, didx, ones_v, acc):
        c = lax.axis_index("c")
        s = lax.axis_index("s")
        w = s * NC + c

        def fill(i, carry):
            ones_v[i] = jnp.ones((16,), jnp.float32)
            return carry
        lax.fori_loop(0, 128, fill, 0)

        pltpu.sync_copy(z_hbm.at[pl.ds(s * RPS, RPS)],
                        acc.at[pl.ds(s * RPS, RPS)])
        pltpu.sync_copy(dst_hbm.at[w], didx)
        plsc.subcore_barrier()

        def body(j, carry):
            pltpu.sync_copy(ones_v, acc.at[didx.at[j]], add=True)
            return carry
        lax.fori_loop(0, KE, body, 0)

        plsc.subcore_barrier()
        pltpu.sync_copy(acc.at[pl.ds(s * RPS, RPS)],
                        out_hbm.at[c, pl.ds(s * RPS, RPS)])

    return k(dst3, z16)


def _sc_aggregate(hs, sd3, zrows):
    @functools.partial(
        pl.kernel,
        out_type=jax.ShapeDtypeStruct((NC, NP, D), jnp.float32),
        mesh=_mesh(),
        scratch_types=[
            pltpu.VMEM((E2 // 2, 256), jnp.int32),
            pltpu.VMEM((CW, D), jnp.float32),
            pltpu.VMEM((CW, D), jnp.float32),
            pltpu.VMEM_SHARED((NP, D), jnp.float32),
            pltpu.SemaphoreType.DMA,
            pltpu.SemaphoreType.DMA,
            pltpu.SemaphoreType.DMA,
            pltpu.SemaphoreType.DMA,
        ],
    )
    def k(hs_hbm, sd_hbm, z_hbm, out_hbm,
          sd, r0, r1, acc, g0, g1, s0, s1):
        rows = (r0, r1)
        gsem = (g0, g1)
        ssem = (s0, s1)
        c = lax.axis_index("c")
        s = lax.axis_index("s")
        w = s * NC + c

        pltpu.sync_copy(z_hbm.at[pl.ds(s * RPS, RPS)],
                        acc.at[pl.ds(s * RPS, RPS)])
        plsc.subcore_barrier()

        PH = E2 // 2

        def gather(j, b):
            pltpu.async_copy(hs_hbm.at[sd.at[j, pl.ds(0, CW)]], rows[b], gsem[b])

        def gwait(j, b):
            pltpu.make_async_copy(
                hs_hbm.at[sd.at[j, pl.ds(0, CW)]], rows[b], gsem[b]).wait()

        def scat(j, b):
            pltpu.async_copy(rows[b], acc.at[sd.at[j, pl.ds(CW, CW)]],
                             ssem[b], add=True)

        def swait(j, b):
            pltpu.make_async_copy(
                rows[b], acc.at[sd.at[j, pl.ds(CW, CW)]], ssem[b]).wait()

        def phase(p, carry):
            pltpu.sync_copy(sd_hbm.at[w, pl.ds(p * PH, PH)], sd)
            gather(0, 0)
            gather(1, 1)

            def duo(i, c2):
                j0 = i * 2
                for kk in range(2):
                    j = j0 + kk
                    gwait(j, kk)
                    scat(j, kk)

                    @pl.when(j + 2 < PH)
                    def _prefetch():
                        swait(j, kk)
                        gather(j + 2, kk)
                return c2
            lax.fori_loop(0, PH // 2, duo, 0)
            swait(PH - 2, 0)
            swait(PH - 1, 1)
            return carry
        lax.fori_loop(0, 2, phase, 0)

        plsc.subcore_barrier()
        pltpu.sync_copy(acc.at[pl.ds(s * RPS, RPS)],
                        out_hbm.at[c, pl.ds(s * RPS, RPS)])

    return k(hs, sd3, zrows)


def _sc_score(h3, qs3, qt3):
    @functools.partial(
        pl.kernel,
        out_type=jax.ShapeDtypeStruct((QP,), jnp.float32),
        mesh=_mesh(),
        scratch_types=[
            pltpu.VMEM((KQ, QC), jnp.int32),
            pltpu.VMEM((KQ, QC), jnp.int32),
            pltpu.VMEM((QC, D), jnp.float32),
            pltpu.VMEM((QC, D), jnp.float32),
            pltpu.VMEM((QC, D), jnp.float32),
            pltpu.VMEM((QC, D), jnp.float32),
            pltpu.VMEM((QC,), jnp.float32),
            pltpu.SemaphoreType.DMA,
            pltpu.SemaphoreType.DMA,
        ],
    )
    def k(h3_hbm, qs_hbm, qt_hbm, out_hbm,
          qsi, qti, xs0, xt0, xs1, xt1, sbuf, sem0, sem1):
        c = lax.axis_index("c")
        s = lax.axis_index("s")
        w = s * NC + c
        base = w * (KQ * QC)
        lanes = lax.broadcasted_iota(jnp.int32, (16,), 0)

        pltpu.sync_copy(qs_hbm.at[w], qsi)
        pltpu.sync_copy(qt_hbm.at[w], qti)

        def start(j, xs, xt, sem):
            pltpu.async_copy(h3_hbm.at[qsi.at[j]], xs, sem)
            pltpu.async_copy(h3_hbm.at[qti.at[j]], xt, sem)

        def drain(j, xs, xt, sem):
            pltpu.make_async_copy(h3_hbm.at[qsi.at[j]], xs, sem).wait()
            pltpu.make_async_copy(h3_hbm.at[qti.at[j]], xt, sem).wait()

        def compute(j, xs, xt):
            def group(g, carry2):
                svec = jnp.zeros((16,), jnp.float32)
                for l in range(16):
                    i = g * 16 + l
                    acc = xs[i, pl.ds(0, 16)] * xt[i, pl.ds(0, 16)]
                    for kk in range(1, 8):
                        acc = acc + (xs[i, pl.ds(kk * 16, 16)]
                                     * xt[i, pl.ds(kk * 16, 16)])
                    for m in (1, 2, 4, 8):
                        acc = acc + _lane_shuffle(acc, lanes ^ m)
                    svec = jnp.where(lanes == l, acc, svec)
                sbuf[pl.ds(g * 16, 16)] = 1.0 / (1.0 + jnp.exp(-svec))
                return carry2
            lax.fori_loop(0, QC // 16, group, 0)
            pltpu.sync_copy(sbuf, out_hbm.at[pl.ds(base + j * QC, QC)])

        H = KQ // 2
        start(0, xs0, xt0, sem0)

        def chunk(jj, carry):
            j0 = jj * 2
            start(j0 + 1, xs1, xt1, sem1)
            drain(j0, xs0, xt0, sem0)
            compute(j0, xs0, xt0)

            @pl.when(jj < H - 1)
            def _prefetch():
                start(j0 + 2, xs0, xt0, sem0)

            drain(j0 + 1, xs1, xt1, sem1)
            compute(j0 + 1, xs1, xt1)
            return carry
        lax.fori_loop(0, H, chunk, 0)

    return k(h3, qs3, qt3)


def _dinv_from(degp0, degp1):
    return lax.rsqrt(degp0[:, 0:1] + degp1[:, 0:1] + 1.0)


def _tc_mm(x_p, W1):
    def body(x_ref, w_ref, out_ref):
        out_ref[...] = jnp.dot(x_ref[...], w_ref[...],
                               preferred_element_type=jnp.float32)

    return pl.pallas_call(
        body,
        out_shape=jax.ShapeDtypeStruct((NP, D), jnp.float32),
    )(x_p, W1)


def _tc_scale(xw, degp):
    def body(xw_ref, degp_ref, out_ref):
        dinv = _dinv_from(degp_ref[0], degp_ref[1])
        out_ref[...] = xw_ref[...] * dinv

    return pl.pallas_call(
        body,
        out_shape=jax.ShapeDtypeStruct((NP, D), jnp.float32),
    )(xw, degp)


def _tc_mid(aggp, hs, degp, b, g, be, Wn):
    def body(aggp_ref, hs_ref, degp_ref, b_ref, g_ref, be_ref, w_ref, out_ref):
        dinv = _dinv_from(degp_ref[0], degp_ref[1])
        pre = (aggp_ref[0] + aggp_ref[1] + hs_ref[...]) * dinv + b_ref[...]
        valid = pre[0:N, :]
        m = jnp.mean(valid, axis=0, keepdims=True)
        v = jnp.mean((valid - m) ** 2, axis=0, keepdims=True)
        hbn = (pre - m) * lax.rsqrt(v + 1e-5) * g_ref[...] + be_ref[...]
        hr = jnp.maximum(hbn, 0.0)
        hn = jnp.dot(hr, w_ref[...], preferred_element_type=jnp.float32) * dinv
        rows = lax.broadcasted_iota(jnp.int32, (NP, 1), 0)
        out_ref[...] = jnp.where(rows < N, hn, 0.0)

    return pl.pallas_call(
        body,
        out_shape=jax.ShapeDtypeStruct((NP, D), jnp.float32),
    )(aggp, hs, degp, b, g, be, Wn)


def _tc_final(aggp, hs, degp, b):
    def body(aggp_ref, hs_ref, degp_ref, b_ref, out_ref):
        dinv = _dinv_from(degp_ref[0], degp_ref[1])
        pre = (aggp_ref[0] + aggp_ref[1] + hs_ref[...]) * dinv + b_ref[...]
        rows = lax.broadcasted_iota(jnp.int32, (NP, 1), 0)
        out_ref[...] = jnp.where(rows < N, pre, 0.0)

    return pl.pallas_call(
        body,
        out_shape=jax.ShapeDtypeStruct((NP, D), jnp.float32),
    )(aggp, hs, degp, b)


def _pad_idx(idx, total):
    pad = total - idx.shape[0]
    fill = N + (jnp.arange(pad, dtype=jnp.int32) % NPAD_ROWS)
    return jnp.concatenate([idx.astype(jnp.int32), fill])


def kernel(x, edge_index, edges, W1, b1, W2, b2, W3, b3, g1, be1, g2, be2):
    src3 = _pad_idx(edge_index[0], EP).reshape(NW, E2, 1, CW)
    dst3 = _pad_idx(edge_index[1], EP).reshape(NW, E2, 1, CW)
    sd3 = jnp.concatenate([src3, dst3], axis=2).reshape(NW, E2, 2 * CW)
    dst3_deg = dst3.reshape(NW, KE, 128)
    qs3 = _pad_idx(edges[0], QP).reshape(NW, KQ, QC)
    qt3 = _pad_idx(edges[1], QP).reshape(NW, KQ, QC)

    x_p = jnp.zeros((NP, D), jnp.float32).at[:N].set(x)
    zrows = jnp.zeros((NP, D), jnp.float32)
    z16 = jnp.zeros((NP, 16), jnp.float32)
    b1r, b2r, b3r = (v.reshape(1, D) for v in (b1, b2, b3))
    g1r, g2r = g1.reshape(1, D), g2.reshape(1, D)
    be1r, be2r = be1.reshape(1, D), be2.reshape(1, D)

    xw1 = _tc_mm(x_p, W1)
    degp = _sc_degree(dst3_deg, z16)

    hs1 = _tc_scale(xw1, degp)
    agg1 = _sc_aggregate(hs1, sd3, zrows)
    hs2 = _tc_mid(agg1, hs1, degp, b1r, g1r, be1r, W2)
    agg2 = _sc_aggregate(hs2, sd3, zrows)
    hs3 = _tc_mid(agg2, hs2, degp, b2r, g2r, be2r, W3)
    agg3 = _sc_aggregate(hs3, sd3, zrows)
    h3 = _tc_final(agg3, hs3, degp, b3r)

    scores = _sc_score(h3, qs3, qt3)
    return scores[:Q]
